# Initial kernel scaffold; baseline (speedup 1.0000x reference)
import functools

import jax
import jax.numpy as jnp
from jax import lax
from jax.experimental import pallas as pl
from jax.experimental.pallas import tpu as pltpu
from jax.experimental.pallas import tpu_sc as plsc

N_ROWS = 320000
D = 128
NCLS = 3
NC = 2
NS = 16
NW = NC * NS
RPW = N_ROWS // NW
CH = 400
NCHUNK = RPW // CH
GRP = CH // 16
LPL = RPW // 16


def _sc_body(feat_hbm, lab_hbm, psum_hbm, pcnt_hbm,
             buf_v, labs_v, outv, cntv, sem_f, sem_l):
  c = lax.axis_index("c")
  s = lax.axis_index("s")
  wid = s * NC + c
  base = wid * RPW

  pltpu.async_copy(lab_hbm.at[pl.ds(base, RPW)], labs_v, sem_l).wait()
  pltpu.async_copy(feat_hbm.at[pl.ds(base, CH)],
                   buf_v.at[pl.ds(0, CH)], sem_f)

  zf = jnp.zeros((16,), jnp.float32)
  init = (zf,) * 26

  def chunk_body(g, carry):
    @pl.when(g + 1 < NCHUNK)
    def _():
      nslot = lax.rem(g + 1, 2)
      pltpu.async_copy(
          feat_hbm.at[pl.ds(base + (g + 1) * CH, CH)],
          buf_v.at[pl.ds(nslot * CH, CH)], sem_f)

    pltpu.make_async_copy(feat_hbm.at[pl.ds(0, CH)],
                          buf_v.at[pl.ds(0, CH)], sem_f).wait()
    rowoff = lax.rem(g, 2) * CH
    labbase = g * CH

    def grp_body(t, cr):
      tot = list(cr[0:8])
      a0 = list(cr[8:16])
      a1 = list(cr[16:24])
      c0 = cr[24]
      c1 = cr[25]
      lab_vec = labs_v[pl.ds(labbase + t * 16, 16)]
      c0 = c0 + (lab_vec == 0).astype(jnp.float32)
      c1 = c1 + (lab_vec == 1).astype(jnp.float32)
      for j in range(16):
        labj = jnp.take_along_axis(
            lab_vec, jnp.full((16,), j, jnp.int32), axis=0,
            mode="promise_in_bounds")
        m0 = (labj == 0).astype(jnp.float32)
        m1 = (labj == 1).astype(jnp.float32)
        r = rowoff + t * 16 + j
        for cc in range(8):
          x = buf_v[r, pl.ds(cc * 16, 16)]
          tot[cc] = tot[cc] + x
          a0[cc] = a0[cc] + x * m0
          a1[cc] = a1[cc] + x * m1
      return tuple(tot) + tuple(a0) + tuple(a1) + (c0, c1)

    return lax.fori_loop(0, GRP, grp_body, carry)

  res = lax.fori_loop(0, NCHUNK, chunk_body, init)
  tot = res[0:8]
  a0 = res[8:16]
  a1 = res[16:24]
  c0 = res[24]
  c1 = res[25]
  for cc in range(8):
    outv[0, pl.ds(cc * 16, 16)] = a0[cc]
    outv[1, pl.ds(cc * 16, 16)] = a1[cc]
    outv[2, pl.ds(cc * 16, 16)] = tot[cc] - a0[cc] - a1[cc]
  cntv[0, :] = c0
  cntv[1, :] = c1
  cntv[2, :] = jnp.full((16,), float(LPL), jnp.float32) - c0 - c1

  pltpu.sync_copy(outv, psum_hbm.at[:, pl.ds(wid * D, D)])
  pltpu.sync_copy(cntv, pcnt_hbm.at[:, pl.ds(wid * 16, 16)])


@functools.partial(
    pl.kernel,
    out_type=(
        jax.ShapeDtypeStruct((NCLS, NW * D), jnp.float32),
        jax.ShapeDtypeStruct((NCLS, NW * 16), jnp.float32),
    ),
    mesh=plsc.VectorSubcoreMesh(core_axis_name="c", subcore_axis_name="s"),
    scratch_types=[
        pltpu.VMEM((2 * CH, D), jnp.float32),
        pltpu.VMEM((RPW,), jnp.int32),
        pltpu.VMEM((NCLS, D), jnp.float32),
        pltpu.VMEM((NCLS, 16), jnp.float32),
        pltpu.SemaphoreType.DMA,
        pltpu.SemaphoreType.DMA,
    ],
)
def _sc_partials(*args):
  _sc_body(*args)


def _finish_body(ps_ref, pc_ref, out_ref):
  sums = jnp.zeros((NCLS, D), jnp.float32)
  for w in range(NW):
    sums = sums + ps_ref[:, w * D:(w + 1) * D]
  cnts = jnp.sum(pc_ref[...], axis=1, keepdims=True)
  centers = sums / cnts
  nrm = jnp.sqrt(jnp.sum(centers * centers, axis=1, keepdims=True))
  out_ref[...] = centers / jnp.maximum(nrm, 1e-12)


def kernel(features, labels):
  psums, pcnts = _sc_partials(features, labels)
  fea_center = pl.pallas_call(
      _finish_body,
      out_shape=jax.ShapeDtypeStruct((NCLS, D), jnp.float32),
  )(psums, pcnts)
  target = jnp.arange(NCLS, dtype=jnp.int32)
  return (fea_center, target)


# speedup vs baseline: 5.1204x; 5.1204x over previous
import functools

import jax
import jax.numpy as jnp
from jax import lax
from jax.experimental import pallas as pl
from jax.experimental.pallas import tpu as pltpu
from jax.experimental.pallas import tpu_sc as plsc

N_ROWS = 320000
D = 128
NCLS = 3
NC = 2
NS = 16
NW = NC * NS
RPW = N_ROWS // NW
CH = 400
NCHUNK = RPW // CH
GRP = CH // 16
LPL = RPW // 16


def _sc_body(feat_hbm, lab_hbm, psum_hbm, pcnt_hbm,
             buf_v, labs_v, outv, cntv, sem_f, sem_l):
  c = lax.axis_index("c")
  s = lax.axis_index("s")
  wid = s * NC + c
  base = wid * RPW

  pltpu.async_copy(lab_hbm.at[pl.ds(base, RPW)], labs_v, sem_l).wait()
  pltpu.async_copy(feat_hbm.at[pl.ds(base, CH)],
                   buf_v.at[pl.ds(0, CH)], sem_f)

  zf = jnp.zeros((16,), jnp.float32)
  init = (zf,) * 26

  def chunk_body(g, carry):
    @pl.when(g + 1 < NCHUNK)
    def _():
      nslot = lax.rem(g + 1, 2)
      pltpu.async_copy(
          feat_hbm.at[pl.ds(base + (g + 1) * CH, CH)],
          buf_v.at[pl.ds(nslot * CH, CH)], sem_f)

    pltpu.make_async_copy(feat_hbm.at[pl.ds(0, CH)],
                          buf_v.at[pl.ds(0, CH)], sem_f).wait()
    rowoff = lax.rem(g, 2) * CH
    labbase = g * CH

    def grp_body(t, cr):
      tot = list(cr[0:8])
      a0 = list(cr[8:16])
      a1 = list(cr[16:24])
      c0 = cr[24]
      c1 = cr[25]
      lab_vec = labs_v[pl.ds(labbase + t * 16, 16)]
      c0 = c0 + (1 - jnp.minimum(lab_vec, 1)).astype(jnp.float32)
      c1 = c1 + (lab_vec & 1).astype(jnp.float32)
      for j in range(16):
        labj = jnp.take_along_axis(
            lab_vec, jnp.full((16,), j, jnp.int32), axis=0,
            mode="promise_in_bounds")
        m0 = (1 - jnp.minimum(labj, 1)).astype(jnp.float32)
        m1 = (labj & 1).astype(jnp.float32)
        r = rowoff + t * 16 + j
        for cc in range(8):
          x = buf_v[r, pl.ds(cc * 16, 16)]
          tot[cc] = tot[cc] + x
          a0[cc] = a0[cc] + x * m0
          a1[cc] = a1[cc] + x * m1
      return tuple(tot) + tuple(a0) + tuple(a1) + (c0, c1)

    return lax.fori_loop(0, GRP, grp_body, carry)

  res = lax.fori_loop(0, NCHUNK, chunk_body, init)
  tot = res[0:8]
  a0 = res[8:16]
  a1 = res[16:24]
  c0 = res[24]
  c1 = res[25]
  for cc in range(8):
    outv[0, pl.ds(cc * 16, 16)] = a0[cc]
    outv[1, pl.ds(cc * 16, 16)] = a1[cc]
    outv[2, pl.ds(cc * 16, 16)] = tot[cc] - a0[cc] - a1[cc]
  cntv[0, pl.ds(0, 16)] = c0
  cntv[1, pl.ds(0, 16)] = c1
  cntv[2, pl.ds(0, 16)] = jnp.full((16,), float(LPL), jnp.float32) - c0 - c1
  for cc in range(1, 8):
    for k in range(NCLS):
      cntv[k, pl.ds(cc * 16, 16)] = zf

  pltpu.sync_copy(outv, psum_hbm.at[:, pl.ds(wid * D, D)])
  pltpu.sync_copy(cntv, pcnt_hbm.at[:, pl.ds(wid * D, D)])


@functools.partial(
    pl.kernel,
    out_type=(
        jax.ShapeDtypeStruct((NCLS, NW * D), jnp.float32),
        jax.ShapeDtypeStruct((NCLS, NW * D), jnp.float32),
    ),
    mesh=plsc.VectorSubcoreMesh(core_axis_name="c", subcore_axis_name="s"),
    scratch_types=[
        pltpu.VMEM((2 * CH, D), jnp.float32),
        pltpu.VMEM((RPW,), jnp.int32),
        pltpu.VMEM((NCLS, D), jnp.float32),
        pltpu.VMEM((NCLS, D), jnp.float32),
        pltpu.SemaphoreType.DMA,
        pltpu.SemaphoreType.DMA,
    ],
)
def _sc_partials(*args):
  _sc_body(*args)


def _finish_body(ps_ref, pc_ref, out_ref):
  sums = jnp.zeros((NCLS, D), jnp.float32)
  for w in range(NW):
    sums = sums + ps_ref[:, w * D:(w + 1) * D]
  cnts = jnp.sum(pc_ref[...], axis=1, keepdims=True)
  centers = sums / cnts
  nrm = jnp.sqrt(jnp.sum(centers * centers, axis=1, keepdims=True))
  out_ref[...] = centers / jnp.maximum(nrm, 1e-12)


def kernel(features, labels):
  psums, pcnts = _sc_partials(features, labels)
  fea_center = pl.pallas_call(
      _finish_body,
      out_shape=jax.ShapeDtypeStruct((NCLS, D), jnp.float32),
  )(psums, pcnts)
  target = jnp.arange(NCLS, dtype=jnp.int32)
  return (fea_center, target)
